# Initial kernel scaffold; baseline (speedup 1.0000x reference)
import jax
import jax.numpy as jnp
from jax.experimental import pallas as pl


def kernel(grad_out, inputs, argmax, batch_size):
    raise NotImplementedError("write your pallas kernel here")


# speedup vs baseline: 18.4481x; 18.4481x over previous
import functools

import jax
import jax.numpy as jnp
from jax import lax
from jax.experimental import pallas as pl
from jax.experimental.pallas import tpu as pltpu
from jax.experimental.pallas import tpu_sc as plsc

B = 8
H = W = 112
C = 96
N = H * W * C
M = 4 * N
NC = 2
NS = 16
LANES = 16
NCHUNK = 4
CHUNK = M // NCHUNK
NT = N // NS
P = 12544
NPIECE = NT // P


def _unpool_body(grad_hbm, arg_hbm, out_hbm, acc, idx_v, val_v, zer_v):
    cid = lax.axis_index("c")
    sid = lax.axis_index("s")

    def fill_zeros(i, _):
        zer_v[pl.ds(i * LANES, LANES)] = jnp.zeros((LANES,), jnp.float32)
        return _

    lax.fori_loop(0, P // LANES, fill_zeros, None)

    def batch_body(b, _):
        def sweep_body(sw, _):
            chunk_id = 2 * sw + cid
            lo = chunk_id * CHUNK

            def zero_body(j, _):
                pltpu.sync_copy(zer_v, acc.at[pl.ds(sid * NT + j * P, P)])
                return _

            lax.fori_loop(0, NPIECE, zero_body, None)
            plsc.subcore_barrier()

            def piece_body(p, _):
                base = b * N + sid * NT + p * P
                pltpu.sync_copy(arg_hbm.at[pl.ds(base, P)], idx_v)
                pltpu.sync_copy(grad_hbm.at[pl.ds(base, P)], val_v)

                def remap_body(i, _):
                    sl = pl.ds(i * LANES, LANES)
                    iv = idx_v[sl]
                    vv = val_v[sl]
                    m = (iv >= lo) & (iv < lo + CHUNK)
                    idx_v[sl] = jnp.where(m, iv - lo, iv >> 2)
                    val_v[sl] = jnp.where(m, vv, 0.0)
                    return _

                lax.fori_loop(0, P // LANES, remap_body, None)
                pltpu.sync_copy(val_v, acc.at[idx_v], add=True)
                return _

            lax.fori_loop(0, NPIECE, piece_body, None)
            plsc.subcore_barrier()

            pltpu.sync_copy(
                acc.at[pl.ds(sid * NT, NT)],
                out_hbm.at[pl.ds(b * M + lo + sid * NT, NT)],
            )
            plsc.subcore_barrier()
            return _

        lax.fori_loop(0, NCHUNK // NC, sweep_body, None)
        return _

    lax.fori_loop(0, B, batch_body, None)


@jax.jit
def _unpool(grad_flat, arg_flat):
    mesh = plsc.VectorSubcoreMesh(core_axis_name="c", subcore_axis_name="s")
    return pl.kernel(
        _unpool_body,
        out_type=jax.ShapeDtypeStruct((B * M,), jnp.float32),
        mesh=mesh,
        scratch_types=[
            pltpu.VMEM_SHARED((CHUNK,), jnp.float32),
            pltpu.VMEM((P,), jnp.int32),
            pltpu.VMEM((P,), jnp.float32),
            pltpu.VMEM((P,), jnp.float32),
        ],
    )(grad_flat, arg_flat)


def kernel(grad_out, inputs, argmax, batch_size):
    del inputs, batch_size
    grad_flat = grad_out.reshape(B * N)
    arg_flat = argmax.reshape(B * N).astype(jnp.int32)
    out_flat = _unpool(grad_flat, arg_flat)
    return out_flat.reshape(B, 2 * H, 2 * W, C)
